# Initial kernel scaffold; baseline (speedup 1.0000x reference)
import functools

import jax
import jax.numpy as jnp
from jax import lax
from jax.experimental import pallas as pl
from jax.experimental.pallas import tpu as pltpu
from jax.experimental.pallas import tpu_sc as plsc

N = 10000
E = 320000
NC = 2
NS = 16
NW = NC * NS
K = 80
E_PER_TILE = E // NW
N_CHUNKS = E_PER_TILE // K
ROWS_PER_TILE = N // NS
ZROWS = 125


def _mm1_body(x_ref, w_ref, o_ref):
    o_ref[...] = jnp.dot(x_ref[...], w_ref[...],
                         preferred_element_type=jnp.float32)


def _layer2_body(p0_ref, p1_ref, b1_ref, w2_ref, o_ref):
    h = jnp.maximum(p0_ref[...] + p1_ref[...] + b1_ref[...], 0.0)
    o_ref[...] = jnp.dot(h, w2_ref[...], preferred_element_type=jnp.float32)


def _final_body(p0_ref, p1_ref, b2_ref, o_ref):
    z = p0_ref[...] + p1_ref[...] + b2_ref[...]
    m = jnp.max(z, axis=1, keepdims=True)
    s = jnp.sum(jnp.exp(z - m), axis=1, keepdims=True)
    o_ref[...] = z - m - jnp.log(s)


def _make_spmm(D: int):
    mesh = plsc.VectorSubcoreMesh(core_axis_name="c", subcore_axis_name="s")

    @functools.partial(
        pl.kernel,
        out_type=jax.ShapeDtypeStruct((NC, N, D), jnp.float32),
        mesh=mesh,
        scratch_types=[
            pltpu.VMEM((N_CHUNKS, K), jnp.int32),
            pltpu.VMEM((N_CHUNKS, K), jnp.int32),
            pltpu.VMEM((N_CHUNKS, K), jnp.float32),
            pltpu.VMEM((K, D), jnp.float32),
            pltpu.VMEM((ZROWS, D), jnp.float32),
            pltpu.VMEM_SHARED((N, D), jnp.float32),
            pltpu.SemaphoreType.DMA,
            pltpu.SemaphoreType.DMA,
        ],
    )
    def spmm(sup, col, row, w, out, col_v, row_v, w_v, rows_v, zbuf, acc,
             gsem, ssem):
        cid = lax.axis_index("c")
        sid = lax.axis_index("s")
        wid = sid * NC + cid

        pltpu.sync_copy(col.at[wid], col_v)
        pltpu.sync_copy(row.at[wid], row_v)
        pltpu.sync_copy(w.at[wid], w_v)

        zero = jnp.zeros((16,), jnp.float32)

        def zfill(r, carry):
            for j in range(D // 16):
                zbuf[r, pl.ds(j * 16, 16)] = zero
            return carry

        lax.fori_loop(0, ZROWS, zfill, 0)
        for i in range(ROWS_PER_TILE // ZROWS):
            pltpu.sync_copy(
                zbuf, acc.at[pl.ds(sid * ROWS_PER_TILE + i * ZROWS, ZROWS)])
        plsc.subcore_barrier()

        def chunk_body(c, carry):
            pltpu.async_copy(sup.at[col_v.at[c]], rows_v, gsem).wait()

            def scale(e, inner):
                wvec = plsc.load_gather(
                    w_v, [jnp.full((16,), c, jnp.int32),
                          jnp.full((16,), e, jnp.int32)])
                for j in range(D // 16):
                    sl = pl.ds(j * 16, 16)
                    rows_v[e, sl] = rows_v[e, sl] * wvec
                return inner

            lax.fori_loop(0, K, scale, 0)
            pltpu.async_copy(rows_v, acc.at[row_v.at[c]], ssem,
                             add=True).wait()
            return carry

        lax.fori_loop(0, N_CHUNKS, chunk_body, 0)
        plsc.subcore_barrier()

        pltpu.sync_copy(
            acc.at[pl.ds(sid * ROWS_PER_TILE, ROWS_PER_TILE)],
            out.at[cid, pl.ds(sid * ROWS_PER_TILE, ROWS_PER_TILE)])

    return spmm


_spmm128 = _make_spmm(128)
_spmm64 = _make_spmm(64)


def kernel(x, edge_index, edge_weight, W1, b1, W2, b2):
    col = edge_index[1].reshape(NW, N_CHUNKS, K)
    row = edge_index[0].reshape(NW, N_CHUNKS, K)
    w3 = edge_weight.reshape(NW, N_CHUNKS, K)

    support1 = pl.pallas_call(
        _mm1_body,
        out_shape=jax.ShapeDtypeStruct((N, 128), jnp.float32),
    )(x, W1)

    parts1 = _spmm128(support1, col, row, w3)

    support2 = pl.pallas_call(
        _layer2_body,
        out_shape=jax.ShapeDtypeStruct((N, 64), jnp.float32),
    )(parts1[0], parts1[1], b1.reshape(1, 128), W2)

    parts2 = _spmm64(support2, col, row, w3)

    out = pl.pallas_call(
        _final_body,
        out_shape=jax.ShapeDtypeStruct((N, 64), jnp.float32),
    )(parts2[0], parts2[1], b2.reshape(1, 64))

    return out


# speedup vs baseline: 4.8084x; 4.8084x over previous
import functools

import jax
import jax.numpy as jnp
from jax import lax
from jax.experimental import pallas as pl
from jax.experimental.pallas import tpu as pltpu
from jax.experimental.pallas import tpu_sc as plsc

N = 10000
E = 320000
NC = 2
NS = 16
NW = NC * NS
K = 80
E_PER_TILE = E // NW
N_CHUNKS = E_PER_TILE // K
GC = 5
NGROUPS = N_CHUNKS // GC
GE = GC * K
NPAD = 10240
ROWS_PER_TILE = NPAD // NS
ZROWS = 128


def _mm1_body(x_ref, w_ref, o_ref):
    o_ref[...] = jnp.dot(x_ref[...], w_ref[...],
                         preferred_element_type=jnp.float32)


def _layer2_body(p_ref, b1_ref, w2_ref, o_ref):
    h = jnp.maximum(p_ref[0, :N, :] + p_ref[1, :N, :] + b1_ref[...], 0.0)
    o_ref[...] = jnp.dot(h, w2_ref[...], preferred_element_type=jnp.float32)


def _final_body(p_ref, b2_ref, o_ref):
    z = p_ref[0, :N, :64] + p_ref[1, :N, :64] + b2_ref[...]
    m = jnp.max(z, axis=1, keepdims=True)
    s = jnp.sum(jnp.exp(z - m), axis=1, keepdims=True)
    o_ref[...] = z - m - jnp.log(s)


@functools.lru_cache(maxsize=None)
def _make_spmm(D: int):
    mesh = plsc.VectorSubcoreMesh(core_axis_name="c", subcore_axis_name="s",
                                  num_cores=NC, num_subcores=NS)

    @functools.partial(
        pl.kernel,
        out_type=jax.ShapeDtypeStruct((NC, NPAD, D), jnp.float32),
        mesh=mesh,
        compiler_params=pltpu.CompilerParams(needs_layout_passes=False),
        scratch_types=[
            pltpu.VMEM((GC, K), jnp.int32),
            pltpu.VMEM((GC, K), jnp.int32),
            pltpu.VMEM((GE,), jnp.float32),
            pltpu.VMEM((K, D), jnp.float32),
            pltpu.VMEM((ZROWS, D), jnp.float32),
            pltpu.VMEM_SHARED((NPAD, D), jnp.float32),
            pltpu.SemaphoreType.DMA,
            pltpu.SemaphoreType.DMA,
        ],
    )
    def spmm(sup, col, row, w, out, col_g, row_g, w_g, rows_v, zbuf, acc,
             gsem, ssem):
        cid = lax.axis_index("c")
        sid = lax.axis_index("s")
        wid = sid * NC + cid

        zero = jnp.zeros((16,), jnp.float32)

        def zfill(r, carry):
            for j in range(D // 16):
                zbuf[r, pl.ds(j * 16, 16)] = zero
            return carry

        lax.fori_loop(0, ZROWS, zfill, 0)
        for i in range(ROWS_PER_TILE // ZROWS):
            pltpu.sync_copy(
                zbuf, acc.at[pl.ds(sid * ROWS_PER_TILE + i * ZROWS, ZROWS)])
        plsc.subcore_barrier()

        def group_body(g, carry):
            pltpu.sync_copy(col.at[wid, g], col_g)
            pltpu.sync_copy(row.at[wid, g], row_g)
            pltpu.sync_copy(w.at[wid, g], w_g)

            def chunk_body(c, carry2):
                pltpu.async_copy(sup.at[col_g.at[c]], rows_v, gsem).wait()

                def scale(e, inner):
                    wvec = plsc.load_gather(
                        w_g, [jnp.full((16,), c * K, jnp.int32) + e])
                    for j in range(D // 16):
                        sl = pl.ds(j * 16, 16)
                        rows_v[e, sl] = rows_v[e, sl] * wvec
                    return inner

                lax.fori_loop(0, K, scale, 0)
                pltpu.async_copy(rows_v, acc.at[row_g.at[c]], ssem,
                                 add=True).wait()
                return carry2

            lax.fori_loop(0, GC, chunk_body, 0)
            return carry

        lax.fori_loop(0, NGROUPS, group_body, 0)
        plsc.subcore_barrier()

        pltpu.sync_copy(
            acc.at[pl.ds(sid * ROWS_PER_TILE, ROWS_PER_TILE)],
            out.at[cid, pl.ds(sid * ROWS_PER_TILE, ROWS_PER_TILE)])

    return spmm


def kernel(x, edge_index, edge_weight, W1, b1, W2, b2):
    col = edge_index[1].reshape(NW, NGROUPS, GC, K)
    row = edge_index[0].reshape(NW, NGROUPS, GC, K)
    w3 = edge_weight.reshape(NW, NGROUPS, GE)

    support1 = pl.pallas_call(
        _mm1_body,
        out_shape=jax.ShapeDtypeStruct((N, 128), jnp.float32),
    )(x, W1)

    parts1 = _make_spmm(128)(support1, col, row, w3)

    W2p = jnp.concatenate([W2, jnp.zeros((128, 64), jnp.float32)], axis=1)
    support2 = pl.pallas_call(
        _layer2_body,
        out_shape=jax.ShapeDtypeStruct((N, 128), jnp.float32),
    )(parts1, b1.reshape(1, 128), W2p)

    parts2 = _make_spmm(128)(support2, col, row, w3)

    out = pl.pallas_call(
        _final_body,
        out_shape=jax.ShapeDtypeStruct((N, 64), jnp.float32),
    )(parts2, b2.reshape(1, 64))

    return out
